# baseline (speedup 1.0000x reference)
import dataclasses
import functools

import jax
import jax.numpy as jnp
from jax import lax
from jax.experimental import pallas as pl
from jax.experimental.pallas import tpu as pltpu
from jax.experimental.pallas import tpu_sc as plsc


def _perm_idx(d):
    import numpy as np

    p = np.arange(d).reshape(d // 32, 2, 16).transpose(0, 2, 1).reshape(d)
    return p


def _i32view(xb):
    n, d = xb.shape
    return lax.bitcast_convert_type(xb.reshape(n, d // 2, 2), jnp.int32)


def _mm_bias(x, w, b, pmat=None, relu=False, block=1000):
    n, d = x.shape
    dout = w.shape[1]
    assert n % block == 0

    def body(x_ref, w_ref, b_ref, p_ref, o_ref, ob_ref):
        y = jnp.dot(x_ref[...], w_ref[...], preferred_element_type=jnp.float32)
        y = y + b_ref[...]
        if relu:
            y = jnp.maximum(y, 0.0)
        o_ref[...] = y
        if ob_ref is not None:
            ob_ref[...] = jnp.dot(
                y, p_ref[...], preferred_element_type=jnp.float32
            ).astype(jnp.bfloat16)

    two = pmat is not None

    def body2(x_ref, w_ref, b_ref, *rest):
        if two:
            p_ref, o_ref, ob_ref = rest
        else:
            (o_ref,) = rest
            p_ref, ob_ref = None, None
        body(x_ref, w_ref, b_ref, p_ref, o_ref, ob_ref)

    in_specs = [
        pl.BlockSpec((block, d), lambda i: (i, 0)),
        pl.BlockSpec((d, dout), lambda i: (0, 0)),
        pl.BlockSpec((1, dout), lambda i: (0, 0)),
    ]
    args = [x, w, b.reshape(1, dout)]
    out_specs = pl.BlockSpec((block, dout), lambda i: (i, 0))
    out_shape = jax.ShapeDtypeStruct((n, dout), jnp.float32)
    if two:
        in_specs.append(pl.BlockSpec((dout, dout), lambda i: (0, 0)))
        args.append(pmat)
        out_specs = [out_specs, pl.BlockSpec((block, dout), lambda i: (i, 0))]
        out_shape = [out_shape, jax.ShapeDtypeStruct((n, dout), jnp.bfloat16)]

    return pl.pallas_call(
        body2,
        grid=(n // block,),
        in_specs=in_specs,
        out_specs=out_specs,
        out_shape=out_shape,
    )(*args)


def _edge_bias(bond, we0, we1, we2, block):
    e, de = bond.shape
    d = we0.shape[1]
    assert e % block == 0

    def body(b_ref, w0_ref, w1_ref, w2_ref, o0_ref, o1_ref, o2_ref):
        bv = b_ref[...]
        for w_ref, o_ref in ((w0_ref, o0_ref), (w1_ref, o1_ref), (w2_ref, o2_ref)):
            o_ref[...] = jnp.dot(
                bv, w_ref[...], preferred_element_type=jnp.float32
            ).astype(jnp.bfloat16)

    w_spec = pl.BlockSpec((de, d), lambda i: (0, 0))
    o_spec = pl.BlockSpec((block, d), lambda i: (i, 0))
    return pl.pallas_call(
        body,
        grid=(e // block,),
        in_specs=[pl.BlockSpec((block, de), lambda i: (i, 0)), w_spec, w_spec, w_spec],
        out_specs=[o_spec, o_spec, o_spec],
        out_shape=[jax.ShapeDtypeStruct((e, d), jnp.bfloat16)] * 3,
    )(bond, we0, we1, we2)


def _dense_update(agg0, agg1, h_in, wg, bg, pmat=None, block=1000):
    n, d = h_in.shape
    assert n % block == 0
    two = pmat is not None

    def body(a0_ref, a1_ref, hin_ref, w_ref, b_ref, *rest):
        if two:
            p_ref, o_ref, ob_ref = rest
        else:
            (o_ref,) = rest
        hin = hin_ref[...]
        t = a0_ref[...] + a1_ref[...] + hin
        t = jnp.dot(t, w_ref[...], preferred_element_type=jnp.float32) + b_ref[...]
        m = jnp.mean(t, axis=-1, keepdims=True)
        v = jnp.mean((t - m) * (t - m), axis=-1, keepdims=True)
        h = (t - m) * lax.rsqrt(v + 1e-5) + hin
        o_ref[...] = h
        if two:
            ob_ref[...] = jnp.dot(
                h, p_ref[...], preferred_element_type=jnp.float32
            ).astype(jnp.bfloat16)

    spec = pl.BlockSpec((block, d), lambda i: (i, 0))
    in_specs = [
        spec,
        spec,
        spec,
        pl.BlockSpec((d, d), lambda i: (0, 0)),
        pl.BlockSpec((1, d), lambda i: (0, 0)),
    ]
    args = [agg0, agg1, h_in, wg, bg.reshape(1, d)]
    out_specs = spec
    out_shape = jax.ShapeDtypeStruct((n, d), jnp.float32)
    if two:
        in_specs.append(pl.BlockSpec((d, d), lambda i: (0, 0)))
        args.append(pmat)
        out_specs = [spec, spec]
        out_shape = [out_shape, jax.ShapeDtypeStruct((n, d), jnp.bfloat16)]

    return pl.pallas_call(
        body,
        grid=(n // block,),
        in_specs=in_specs,
        out_specs=out_specs,
        out_shape=out_shape,
    )(*args)


def _hin_update(h, vn, batch3, pmat, block=400):
    n, d = h.shape
    g = vn.shape[0]
    assert n % block == 0

    def body(b_ref, h_ref, vn_ref, p_ref, o_ref, ob_ref):
        bv = b_ref[...].reshape(block)
        onehot = (bv[:, None] == lax.broadcasted_iota(jnp.int32, (block, g), 1))
        onehot = onehot.astype(jnp.float32)
        hin = h_ref[...] + jnp.dot(onehot, vn_ref[...], preferred_element_type=jnp.float32)
        o_ref[...] = hin
        ob_ref[...] = jnp.dot(
            hin, p_ref[...], preferred_element_type=jnp.float32
        ).astype(jnp.bfloat16)

    spec = pl.BlockSpec((block, d), lambda i: (i, 0))
    return pl.pallas_call(
        body,
        grid=(n // block,),
        in_specs=[
            pl.BlockSpec((1, 1, block), lambda i: (i, 0, 0)),
            spec,
            pl.BlockSpec((g, d), lambda i: (0, 0)),
            pl.BlockSpec((d, d), lambda i: (0, 0)),
        ],
        out_specs=[spec, spec],
        out_shape=[
            jax.ShapeDtypeStruct((n, d), jnp.float32),
            jax.ShapeDtypeStruct((n, d), jnp.bfloat16),
        ],
    )(batch3, h, vn, pmat)


def _vn_update(h_in, vn, batch3, w1, b1, w2, b2, block=400):
    n, d = h_in.shape
    g = vn.shape[0]
    d2 = w1.shape[1]
    nb = n // block
    assert n % block == 0

    def body(b_ref, hin_ref, vn_ref, w1_ref, b1_ref, w2_ref, b2_ref, o_ref, acc):
        i = pl.program_id(0)

        @pl.when(i == 0)
        def _():
            acc[...] = jnp.zeros_like(acc)

        bv = b_ref[...].reshape(block)
        onehot = (lax.broadcasted_iota(jnp.int32, (g, block), 0) == bv[None, :])
        onehot = onehot.astype(jnp.float32)
        acc[...] += jnp.dot(onehot, hin_ref[...], preferred_element_type=jnp.float32)

        @pl.when(i == nb - 1)
        def _():
            p = acc[...] + vn_ref[...]
            t = jnp.dot(p, w1_ref[...], preferred_element_type=jnp.float32) + b1_ref[...]
            m = jnp.mean(t, axis=0, keepdims=True)
            v = jnp.mean((t - m) * (t - m), axis=0, keepdims=True)
            t = jnp.maximum((t - m) * lax.rsqrt(v + 1e-5), 0.0)
            t = jnp.dot(t, w2_ref[...], preferred_element_type=jnp.float32) + b2_ref[...]
            m = jnp.mean(t, axis=0, keepdims=True)
            v = jnp.mean((t - m) * (t - m), axis=0, keepdims=True)
            t = jnp.maximum((t - m) * lax.rsqrt(v + 1e-5), 0.0)
            o_ref[...] = vn_ref[...] + t

    return pl.pallas_call(
        body,
        grid=(nb,),
        in_specs=[
            pl.BlockSpec((1, 1, block), lambda i: (i, 0, 0)),
            pl.BlockSpec((block, d), lambda i: (i, 0)),
            pl.BlockSpec((g, d), lambda i: (0, 0)),
            pl.BlockSpec((d, d2), lambda i: (0, 0)),
            pl.BlockSpec((1, d2), lambda i: (0, 0)),
            pl.BlockSpec((d2, d), lambda i: (0, 0)),
            pl.BlockSpec((1, d), lambda i: (0, 0)),
        ],
        out_specs=pl.BlockSpec((g, d), lambda i: (0, 0)),
        out_shape=jax.ShapeDtypeStruct((g, d), jnp.float32),
        scratch_shapes=[pltpu.VMEM((g, d), jnp.float32)],
    )(batch3, h_in, vn, w1, b1.reshape(1, d2), w2, b2.reshape(1, d))


_NC = 2
_NS = 16
_CHUNK = 72


def _sc_pad_shapes(n, e):
    nw = _NC * _NS
    rows_per_tile = -(-n // _NS)
    zrows = -(-rows_per_tile // _CHUNK) * _CHUNK
    n_pad = _NS * zrows
    nchunk = -(-e // (nw * _CHUNK))
    while nchunk % 4 != 1:
        nchunk += 1
    return n_pad, nchunk


def _sc_edge_agg(hb, eb, src3, dst3):
    n = hb.shape[0]
    d = hb.shape[1] * 2
    nw, nchunk, _ = src3.shape
    ep = nchunk * _CHUNK
    assert nw == _NC * _NS
    n_pad, _ = _sc_pad_shapes(n, nw * ep)
    zrows = n_pad // _NS
    assert zrows % _CHUNK == 0 and nchunk % 4 == 1

    mesh = plsc.VectorSubcoreMesh(core_axis_name="c", subcore_axis_name="s")
    cp = pltpu.CompilerParams(
        needs_layout_passes=False, use_tc_tiling_on_sc=False
    )

    @functools.partial(
        pl.kernel,
        out_type=jax.ShapeDtypeStruct((_NC, n_pad, d), jnp.float32),
        mesh=mesh,
        compiler_params=cp,
        scratch_types=[
            [pltpu.VMEM((_CHUNK,), jnp.int32) for _ in range(4)],
            [pltpu.VMEM((_CHUNK,), jnp.int32) for _ in range(4)],
            [pltpu.VMEM((_CHUNK, d // 2), jnp.int32) for _ in range(2)],
            [pltpu.VMEM((_CHUNK, d // 2), jnp.int32) for _ in range(2)],
            [pltpu.VMEM((_CHUNK, d), jnp.float32) for _ in range(2)],
            pltpu.VMEM_SHARED((n_pad, d), jnp.float32),
            [pltpu.SemaphoreType.DMA for _ in range(4)],
            [pltpu.SemaphoreType.DMA for _ in range(2)],
            [pltpu.SemaphoreType.DMA for _ in range(2)],
            [pltpu.SemaphoreType.DMA for _ in range(2)],
        ],
    )
    def k(hb_hbm, eb_hbm, src_hbm, dst_hbm, out_hbm, sidx, didx,
          rows, ebv, obuf, acc, si, sg, se, ss):
        c = lax.axis_index("c")
        s = lax.axis_index("s")
        wid = s * _NC + c
        ebase = wid * ep

        def issue_idx(ci, q):
            pltpu.async_copy(src_hbm.at[wid, ci], sidx[q], si[q])
            pltpu.async_copy(dst_hbm.at[wid, ci], didx[q], si[q])

        def wait_idx(q):
            pltpu.make_async_copy(src_hbm.at[wid, 0], sidx[q], si[q]).wait()
            pltpu.make_async_copy(dst_hbm.at[wid, 0], didx[q], si[q]).wait()

        def issue_gather(ci, p, q):
            pltpu.async_copy(hb_hbm.at[sidx[q]], rows[p], sg[p])
            pltpu.async_copy(eb_hbm.at[pl.ds(ebase + ci * _CHUNK, _CHUNK)],
                             ebv[p], se[p])

        def wait_gather(p):
            pltpu.make_async_copy(hb_hbm.at[sidx[0]], rows[p], sg[p]).wait()
            pltpu.make_async_copy(eb_hbm.at[pl.ds(ebase, _CHUNK)],
                                  ebv[p], se[p]).wait()

        def wait_scat(p):
            pltpu.make_async_copy(obuf[p], acc.at[didx[0]], ss[p]).wait()

        def compute(p):
            rp = rows[p]
            ep_ = ebv[p]
            op = obuf[p]
            himask = jnp.int32(-65536)

            @pl.loop(0, _CHUNK, unroll=2)
            def _(r):
                for j in range(d // 32):
                    slw = pl.ds(j * 16, 16)
                    a = plsc.bitcast(rp[r, slw], jnp.bfloat16)
                    b = plsc.bitcast(ep_[r, slw], jnp.bfloat16)
                    m = jnp.maximum(a + b, jnp.bfloat16(0))
                    w = plsc.bitcast(m, jnp.int32)
                    op[r, pl.ds(j * 32, 16)] = plsc.bitcast(w << 16, jnp.float32)
                    op[r, pl.ds(j * 32 + 16, 16)] = plsc.bitcast(
                        w & himask, jnp.float32)

        issue_idx(0, 0)
        issue_idx(1, 1)

        @pl.loop(0, _CHUNK)
        def _(r):
            for j in range(d // 16):
                obuf[0][r, pl.ds(j * 16, 16)] = jnp.zeros((16,), jnp.float32)

        @pl.loop(0, zrows, step=_CHUNK)
        def _(r0):
            pltpu.sync_copy(obuf[0], acc.at[pl.ds(s * zrows + r0, _CHUNK)])

        plsc.subcore_barrier()

        wait_idx(0)
        issue_gather(0, 0, 0)
        wait_idx(1)
        issue_gather(1, 1, 1)

        def step(ci, u):
            p, q, q2 = u % 2, u % 4, (u + 2) % 4
            cc = ci + u
            wait_gather(p)

            @pl.when(cc >= 2)
            def _():
                wait_scat(p)

            @pl.when(cc + 2 < nchunk)
            def _():
                issue_idx(cc + 2, q2)

            compute(p)
            pltpu.async_copy(obuf[p], acc.at[didx[q]], ss[p], add=True)

            @pl.when(cc + 2 < nchunk)
            def _():
                wait_idx(q2)
                issue_gather(cc + 2, p, q2)

        @pl.loop(0, nchunk - 1, step=4)
        def _(ci):
            for u in range(4):
                step(ci, u)

        wait_gather(0)
        wait_scat(0)
        compute(0)
        pltpu.async_copy(obuf[0], acc.at[didx[0]], ss[0], add=True)
        wait_scat(0)
        wait_scat(1)

        plsc.subcore_barrier()
        pltpu.sync_copy(
            acc.at[pl.ds(s * zrows, zrows)], out_hbm.at[c, pl.ds(s * zrows, zrows)]
        )

    return k(hb, eb, src3, dst3)


def kernel(x, edge_index, bond_feature, edge_attr, peripheral_attr, rd, batch,
           W_init, b_init, We0, Wg0, bg0, We1, Wg1, bg1, We2, Wg2, bg2,
           Wv1_0, bv1_0, Wv2_0, bv2_0, Wv1_1, bv1_1, Wv2_1, bv2_1,
           W_out, b_out):
    n, d = x.shape
    g = 512
    nw = _NC * _NS
    e = edge_index.shape[1]
    n_pad, nchunk = _sc_pad_shapes(n, e)
    e_pad = nw * nchunk * _CHUNK
    npad_e = e_pad - e
    ar = jnp.arange(npad_e, dtype=jnp.int32)
    src_p = jnp.concatenate([edge_index[0], ar % n])
    dst_p = jnp.concatenate([edge_index[1], n + ar % (n_pad - n)])
    src3 = src_p.reshape(nw, nchunk, _CHUNK)
    dst3 = dst_p.reshape(nw, nchunk, _CHUNK)
    bond_p = jnp.concatenate(
        [bond_feature, jnp.zeros((npad_e, bond_feature.shape[1]), jnp.float32)]
    )
    batch3 = batch.reshape(n // 400, 1, 400)

    perm = _perm_idx(d)
    pmat = jnp.zeros((d, d), jnp.float32).at[perm, jnp.arange(d)].set(1.0)

    h0, hb = _mm_bias(x, W_init, b_init, pmat=pmat)
    eb0, eb1, eb2 = _edge_bias(
        bond_p, We0[:, perm], We1[:, perm], We2[:, perm], block=e_pad // nchunk
    )
    eb0, eb1, eb2 = _i32view(eb0), _i32view(eb1), _i32view(eb2)

    wgs = (Wg0, Wg1, Wg2)
    bgs = (bg0, bg1, bg2)
    ebs = (eb0, eb1, eb2)
    wv1 = (Wv1_0, Wv1_1)
    bv1 = (bv1_0, bv1_1)
    wv2 = (Wv2_0, Wv2_1)
    bv2 = (bv2_0, bv2_1)

    vn = jnp.zeros((g, d), dtype=jnp.float32)
    h_in = h0
    for l in range(3):
        agg = _sc_edge_agg(_i32view(hb), ebs[l], src3, dst3)
        if l < 2:
            h = _dense_update(agg[0], agg[1], h_in, wgs[l], bgs[l])
            vn = _vn_update(h_in, vn, batch3, wv1[l], bv1[l], wv2[l], bv2[l])
            h_in, hb = _hin_update(h, vn, batch3, pmat)
        else:
            h_in = _dense_update(agg[0], agg[1], h_in, wgs[l], bgs[l])

    return _mm_bias(h_in, W_out, b_out, relu=True)


# speedup vs baseline: 2.4791x; 2.4791x over previous
import dataclasses
import functools

import jax
import jax.numpy as jnp
from jax import lax
from jax.experimental import pallas as pl
from jax.experimental.pallas import tpu as pltpu
from jax.experimental.pallas import tpu_sc as plsc


def _lohi_idx(d):
    import numpy as np

    w = np.arange(d // 2)
    lo = 32 * (w // 16) + w % 16
    return lo, lo + 16


def _pack_bf16_pair(a, b):
    ua = lax.bitcast_convert_type(a, jnp.uint32)
    ub = lax.bitcast_convert_type(b, jnp.uint32)
    one = jnp.uint32(1)
    half = jnp.uint32(0x7FFF)
    bfa = (ua + half + ((ua >> 16) & one)) >> 16
    bfb = (ub + half + ((ub >> 16) & one)) >> 16
    return lax.bitcast_convert_type(bfa | (bfb << 16), jnp.int32)


def _mm_bias(x, w, b, pmat=None, relu=False, block=1000):
    n, d = x.shape
    dout = w.shape[1]
    assert n % block == 0

    two = pmat is not None

    def body2(x_ref, w_ref, b_ref, *rest):
        if two:
            plo_ref, phi_ref, o_ref, ob_ref = rest
        else:
            (o_ref,) = rest
        y = jnp.dot(x_ref[...], w_ref[...], preferred_element_type=jnp.float32)
        y = y + b_ref[...]
        if relu:
            y = jnp.maximum(y, 0.0)
        o_ref[...] = y
        if two:
            a = jnp.dot(y, plo_ref[...], preferred_element_type=jnp.float32)
            bb = jnp.dot(y, phi_ref[...], preferred_element_type=jnp.float32)
            ob_ref[...] = _pack_bf16_pair(a, bb)

    in_specs = [
        pl.BlockSpec((block, d), lambda i: (i, 0)),
        pl.BlockSpec((d, dout), lambda i: (0, 0)),
        pl.BlockSpec((1, dout), lambda i: (0, 0)),
    ]
    args = [x, w, b.reshape(1, dout)]
    out_specs = pl.BlockSpec((block, dout), lambda i: (i, 0))
    out_shape = jax.ShapeDtypeStruct((n, dout), jnp.float32)
    if two:
        half_spec = pl.BlockSpec((dout, dout // 2), lambda i: (0, 0))
        in_specs += [half_spec, half_spec]
        args += list(pmat)
        out_specs = [out_specs, pl.BlockSpec((block, dout // 2), lambda i: (i, 0))]
        out_shape = [out_shape, jax.ShapeDtypeStruct((n, dout // 2), jnp.int32)]

    return pl.pallas_call(
        body2,
        grid=(n // block,),
        in_specs=in_specs,
        out_specs=out_specs,
        out_shape=out_shape,
    )(*args)


def _edge_bias(bond, wes, block):
    e, de = bond.shape
    dh = wes[0].shape[1]
    assert e % block == 0

    def body(b_ref, *rest):
        w_refs, o_refs = rest[:6], rest[6:]
        bv = b_ref[...]
        for i in range(3):
            a = jnp.dot(bv, w_refs[2 * i][...], preferred_element_type=jnp.float32)
            bb = jnp.dot(bv, w_refs[2 * i + 1][...], preferred_element_type=jnp.float32)
            o_refs[i][...] = _pack_bf16_pair(a, bb)

    w_spec = pl.BlockSpec((de, dh), lambda i: (0, 0))
    o_spec = pl.BlockSpec((block, dh), lambda i: (i, 0))
    return pl.pallas_call(
        body,
        grid=(e // block,),
        in_specs=[pl.BlockSpec((block, de), lambda i: (i, 0))] + [w_spec] * 6,
        out_specs=[o_spec, o_spec, o_spec],
        out_shape=[jax.ShapeDtypeStruct((e, dh), jnp.int32)] * 3,
    )(bond, *wes)


def _dense_update(agg0, agg1, h_in, wg, bg, pmat=None, block=1000):
    n, d = h_in.shape
    assert n % block == 0
    two = pmat is not None

    def body(a0_ref, a1_ref, hin_ref, w_ref, b_ref, *rest):
        if two:
            plo_ref, phi_ref, o_ref, ob_ref = rest
        else:
            (o_ref,) = rest
        hin = hin_ref[...]
        t = a0_ref[...] + a1_ref[...] + hin
        t = jnp.dot(t, w_ref[...], preferred_element_type=jnp.float32) + b_ref[...]
        m = jnp.mean(t, axis=-1, keepdims=True)
        v = jnp.mean((t - m) * (t - m), axis=-1, keepdims=True)
        h = (t - m) * lax.rsqrt(v + 1e-5) + hin
        o_ref[...] = h
        if two:
            a = jnp.dot(h, plo_ref[...], preferred_element_type=jnp.float32)
            bb = jnp.dot(h, phi_ref[...], preferred_element_type=jnp.float32)
            ob_ref[...] = _pack_bf16_pair(a, bb)

    spec = pl.BlockSpec((block, d), lambda i: (i, 0))
    in_specs = [
        spec,
        spec,
        spec,
        pl.BlockSpec((d, d), lambda i: (0, 0)),
        pl.BlockSpec((1, d), lambda i: (0, 0)),
    ]
    args = [agg0, agg1, h_in, wg, bg.reshape(1, d)]
    out_specs = spec
    out_shape = jax.ShapeDtypeStruct((n, d), jnp.float32)
    if two:
        half_spec = pl.BlockSpec((d, d // 2), lambda i: (0, 0))
        in_specs += [half_spec, half_spec]
        args += list(pmat)
        out_specs = [spec, pl.BlockSpec((block, d // 2), lambda i: (i, 0))]
        out_shape = [out_shape, jax.ShapeDtypeStruct((n, d // 2), jnp.int32)]

    return pl.pallas_call(
        body,
        grid=(n // block,),
        in_specs=in_specs,
        out_specs=out_specs,
        out_shape=out_shape,
    )(*args)


def _hin_update(h, vn, batch3, pmat, block=400):
    n, d = h.shape
    g = vn.shape[0]
    assert n % block == 0

    def body(b_ref, h_ref, vn_ref, plo_ref, phi_ref, o_ref, ob_ref):
        bv = b_ref[...].reshape(block)
        onehot = (bv[:, None] == lax.broadcasted_iota(jnp.int32, (block, g), 1))
        onehot = onehot.astype(jnp.float32)
        hin = h_ref[...] + jnp.dot(onehot, vn_ref[...], preferred_element_type=jnp.float32)
        o_ref[...] = hin
        a = jnp.dot(hin, plo_ref[...], preferred_element_type=jnp.float32)
        bb = jnp.dot(hin, phi_ref[...], preferred_element_type=jnp.float32)
        ob_ref[...] = _pack_bf16_pair(a, bb)

    spec = pl.BlockSpec((block, d), lambda i: (i, 0))
    half_spec = pl.BlockSpec((d, d // 2), lambda i: (0, 0))
    return pl.pallas_call(
        body,
        grid=(n // block,),
        in_specs=[
            pl.BlockSpec((1, 1, block), lambda i: (i, 0, 0)),
            spec,
            pl.BlockSpec((g, d), lambda i: (0, 0)),
            half_spec,
            half_spec,
        ],
        out_specs=[spec, pl.BlockSpec((block, d // 2), lambda i: (i, 0))],
        out_shape=[
            jax.ShapeDtypeStruct((n, d), jnp.float32),
            jax.ShapeDtypeStruct((n, d // 2), jnp.int32),
        ],
    )(batch3, h, vn, *pmat)


def _vn_update(h_in, vn, batch3, w1, b1, w2, b2, block=400):
    n, d = h_in.shape
    g = vn.shape[0]
    d2 = w1.shape[1]
    nb = n // block
    assert n % block == 0

    def body(b_ref, hin_ref, vn_ref, w1_ref, b1_ref, w2_ref, b2_ref, o_ref, acc):
        i = pl.program_id(0)

        @pl.when(i == 0)
        def _():
            acc[...] = jnp.zeros_like(acc)

        bv = b_ref[...].reshape(block)
        onehot = (lax.broadcasted_iota(jnp.int32, (g, block), 0) == bv[None, :])
        onehot = onehot.astype(jnp.float32)
        acc[...] += jnp.dot(onehot, hin_ref[...], preferred_element_type=jnp.float32)

        @pl.when(i == nb - 1)
        def _():
            p = acc[...] + vn_ref[...]
            t = jnp.dot(p, w1_ref[...], preferred_element_type=jnp.float32) + b1_ref[...]
            m = jnp.mean(t, axis=0, keepdims=True)
            v = jnp.mean((t - m) * (t - m), axis=0, keepdims=True)
            t = jnp.maximum((t - m) * lax.rsqrt(v + 1e-5), 0.0)
            t = jnp.dot(t, w2_ref[...], preferred_element_type=jnp.float32) + b2_ref[...]
            m = jnp.mean(t, axis=0, keepdims=True)
            v = jnp.mean((t - m) * (t - m), axis=0, keepdims=True)
            t = jnp.maximum((t - m) * lax.rsqrt(v + 1e-5), 0.0)
            o_ref[...] = vn_ref[...] + t

    return pl.pallas_call(
        body,
        grid=(nb,),
        in_specs=[
            pl.BlockSpec((1, 1, block), lambda i: (i, 0, 0)),
            pl.BlockSpec((block, d), lambda i: (i, 0)),
            pl.BlockSpec((g, d), lambda i: (0, 0)),
            pl.BlockSpec((d, d2), lambda i: (0, 0)),
            pl.BlockSpec((1, d2), lambda i: (0, 0)),
            pl.BlockSpec((d2, d), lambda i: (0, 0)),
            pl.BlockSpec((1, d), lambda i: (0, 0)),
        ],
        out_specs=pl.BlockSpec((g, d), lambda i: (0, 0)),
        out_shape=jax.ShapeDtypeStruct((g, d), jnp.float32),
        scratch_shapes=[pltpu.VMEM((g, d), jnp.float32)],
    )(batch3, h_in, vn, w1, b1.reshape(1, d2), w2, b2.reshape(1, d))


_NC = 2
_NS = 16
_CHUNK = 72


def _sc_pad_shapes(n, e):
    nw = _NC * _NS
    rows_per_tile = -(-n // _NS)
    zrows = -(-rows_per_tile // _CHUNK) * _CHUNK
    n_pad = _NS * zrows
    nchunk = -(-e // (nw * _CHUNK))
    while nchunk % 4 != 1:
        nchunk += 1
    return n_pad, nchunk


def _sc_edge_agg(hb, eb, src3, dst3):
    n = hb.shape[0]
    d = hb.shape[1] * 2
    nw, nchunk, _ = src3.shape
    ep = nchunk * _CHUNK
    assert nw == _NC * _NS
    n_pad, _ = _sc_pad_shapes(n, nw * ep)
    zrows = n_pad // _NS
    assert zrows % _CHUNK == 0 and nchunk % 4 == 1

    mesh = plsc.VectorSubcoreMesh(core_axis_name="c", subcore_axis_name="s")
    cp = pltpu.CompilerParams(
        needs_layout_passes=False, use_tc_tiling_on_sc=False
    )

    @functools.partial(
        pl.kernel,
        out_type=jax.ShapeDtypeStruct((_NC, n_pad, d), jnp.float32),
        mesh=mesh,
        compiler_params=cp,
        scratch_types=[
            [pltpu.VMEM((_CHUNK,), jnp.int32) for _ in range(4)],
            [pltpu.VMEM((_CHUNK,), jnp.int32) for _ in range(4)],
            [pltpu.VMEM((_CHUNK, d // 2), jnp.int32) for _ in range(2)],
            [pltpu.VMEM((_CHUNK, d // 2), jnp.int32) for _ in range(2)],
            [pltpu.VMEM((_CHUNK, d), jnp.float32) for _ in range(2)],
            pltpu.VMEM_SHARED((n_pad, d), jnp.float32),
            [pltpu.SemaphoreType.DMA for _ in range(4)],
            [pltpu.SemaphoreType.DMA for _ in range(2)],
            [pltpu.SemaphoreType.DMA for _ in range(2)],
            [pltpu.SemaphoreType.DMA for _ in range(2)],
        ],
    )
    def k(hb_hbm, eb_hbm, src_hbm, dst_hbm, out_hbm, sidx, didx,
          rows, ebv, obuf, acc, si, sg, se, ss):
        c = lax.axis_index("c")
        s = lax.axis_index("s")
        wid = s * _NC + c
        ebase = wid * ep

        def issue_idx(ci, q):
            pltpu.async_copy(src_hbm.at[wid, ci], sidx[q], si[q])
            pltpu.async_copy(dst_hbm.at[wid, ci], didx[q], si[q])

        def wait_idx(q):
            pltpu.make_async_copy(src_hbm.at[wid, 0], sidx[q], si[q]).wait()
            pltpu.make_async_copy(dst_hbm.at[wid, 0], didx[q], si[q]).wait()

        def issue_gather(ci, p, q):
            pltpu.async_copy(hb_hbm.at[sidx[q]], rows[p], sg[p])
            pltpu.async_copy(eb_hbm.at[pl.ds(ebase + ci * _CHUNK, _CHUNK)],
                             ebv[p], se[p])

        def wait_gather(p):
            pltpu.make_async_copy(hb_hbm.at[sidx[0]], rows[p], sg[p]).wait()
            pltpu.make_async_copy(eb_hbm.at[pl.ds(ebase, _CHUNK)],
                                  ebv[p], se[p]).wait()

        def wait_scat(p):
            pltpu.make_async_copy(obuf[p], acc.at[didx[0]], ss[p]).wait()

        def compute(p):
            rp = rows[p]
            ep_ = ebv[p]
            op = obuf[p]
            himask = jnp.int32(-65536)

            @pl.loop(0, _CHUNK, unroll=2)
            def _(r):
                for j in range(d // 32):
                    slw = pl.ds(j * 16, 16)
                    a = plsc.bitcast(rp[r, slw], jnp.bfloat16)
                    b = plsc.bitcast(ep_[r, slw], jnp.bfloat16)
                    m = jnp.maximum(a + b, jnp.bfloat16(0))
                    w = plsc.bitcast(m, jnp.int32)
                    op[r, pl.ds(j * 32, 16)] = plsc.bitcast(w << 16, jnp.float32)
                    op[r, pl.ds(j * 32 + 16, 16)] = plsc.bitcast(
                        w & himask, jnp.float32)

        issue_idx(0, 0)
        issue_idx(1, 1)

        @pl.loop(0, _CHUNK)
        def _(r):
            for j in range(d // 16):
                obuf[0][r, pl.ds(j * 16, 16)] = jnp.zeros((16,), jnp.float32)

        @pl.loop(0, zrows, step=_CHUNK)
        def _(r0):
            pltpu.sync_copy(obuf[0], acc.at[pl.ds(s * zrows + r0, _CHUNK)])

        plsc.subcore_barrier()

        wait_idx(0)
        issue_gather(0, 0, 0)
        wait_idx(1)
        issue_gather(1, 1, 1)

        def step(ci, u):
            p, q, q2 = u % 2, u % 4, (u + 2) % 4
            cc = ci + u
            wait_gather(p)

            @pl.when(cc >= 2)
            def _():
                wait_scat(p)

            @pl.when(cc + 2 < nchunk)
            def _():
                issue_idx(cc + 2, q2)

            compute(p)
            pltpu.async_copy(obuf[p], acc.at[didx[q]], ss[p], add=True)

            @pl.when(cc + 2 < nchunk)
            def _():
                wait_idx(q2)
                issue_gather(cc + 2, p, q2)

        @pl.loop(0, nchunk - 1, step=4)
        def _(ci):
            for u in range(4):
                step(ci, u)

        wait_gather(0)
        wait_scat(0)
        compute(0)
        pltpu.async_copy(obuf[0], acc.at[didx[0]], ss[0], add=True)
        wait_scat(0)
        wait_scat(1)

        plsc.subcore_barrier()
        pltpu.sync_copy(
            acc.at[pl.ds(s * zrows, zrows)], out_hbm.at[c, pl.ds(s * zrows, zrows)]
        )

    return k(hb, eb, src3, dst3)


def kernel(x, edge_index, bond_feature, edge_attr, peripheral_attr, rd, batch,
           W_init, b_init, We0, Wg0, bg0, We1, Wg1, bg1, We2, Wg2, bg2,
           Wv1_0, bv1_0, Wv2_0, bv2_0, Wv1_1, bv1_1, Wv2_1, bv2_1,
           W_out, b_out):
    n, d = x.shape
    g = 512
    nw = _NC * _NS
    e = edge_index.shape[1]
    n_pad, nchunk = _sc_pad_shapes(n, e)
    e_pad = nw * nchunk * _CHUNK
    npad_e = e_pad - e
    ar = jnp.arange(npad_e, dtype=jnp.int32)
    src_p = jnp.concatenate([edge_index[0], ar % n])
    dst_p = jnp.concatenate([edge_index[1], n + ar % (n_pad - n)])
    src3 = src_p.reshape(nw, nchunk, _CHUNK)
    dst3 = dst_p.reshape(nw, nchunk, _CHUNK)
    bond_p = jnp.concatenate(
        [bond_feature, jnp.zeros((npad_e, bond_feature.shape[1]), jnp.float32)]
    )
    batch3 = batch.reshape(n // 400, 1, 400)

    lo_idx, hi_idx = _lohi_idx(d)
    eye = jnp.eye(d, dtype=jnp.float32)
    pmat = (eye[:, lo_idx], eye[:, hi_idx])

    h0, hb = _mm_bias(x, W_init, b_init, pmat=pmat)
    wes = (We0[:, lo_idx], We0[:, hi_idx], We1[:, lo_idx], We1[:, hi_idx],
           We2[:, lo_idx], We2[:, hi_idx])
    eb0, eb1, eb2 = _edge_bias(bond_p, wes, block=e_pad // nchunk)

    wgs = (Wg0, Wg1, Wg2)
    bgs = (bg0, bg1, bg2)
    ebs = (eb0, eb1, eb2)
    wv1 = (Wv1_0, Wv1_1)
    bv1 = (bv1_0, bv1_1)
    wv2 = (Wv2_0, Wv2_1)
    bv2 = (bv2_0, bv2_1)

    vn = jnp.zeros((g, d), dtype=jnp.float32)
    h_in = h0
    for l in range(3):
        agg = _sc_edge_agg(hb, ebs[l], src3, dst3)
        if l < 2:
            h = _dense_update(agg[0], agg[1], h_in, wgs[l], bgs[l])
            vn = _vn_update(h_in, vn, batch3, wv1[l], bv1[l], wv2[l], bv2[l])
            h_in, hb = _hin_update(h, vn, batch3, pmat)
        else:
            h_in = _dense_update(agg[0], agg[1], h_in, wgs[l], bgs[l])

    return _mm_bias(h_in, W_out, b_out, relu=True)


# speedup vs baseline: 2.6735x; 1.0784x over previous
import dataclasses
import functools

import jax
import jax.numpy as jnp
from jax import lax
from jax.experimental import pallas as pl
from jax.experimental.pallas import tpu as pltpu
from jax.experimental.pallas import tpu_sc as plsc


def _lohi_idx(d):
    import numpy as np

    w = np.arange(d // 2)
    lo = 32 * (w // 16) + w % 16
    return lo, lo + 16


def _pack_bf16_pair(a, b):
    ua = lax.bitcast_convert_type(a, jnp.uint32)
    ub = lax.bitcast_convert_type(b, jnp.uint32)
    one = jnp.uint32(1)
    half = jnp.uint32(0x7FFF)
    bfa = (ua + half + ((ua >> 16) & one)) >> 16
    bfb = (ub + half + ((ub >> 16) & one)) >> 16
    return lax.bitcast_convert_type(bfa | (bfb << 16), jnp.int32)


def _mm_bias(x, w, b, pmat=None, relu=False, block=1000):
    n, d = x.shape
    dout = w.shape[1]
    assert n % block == 0

    two = pmat is not None

    def body2(x_ref, w_ref, b_ref, *rest):
        if two:
            plo_ref, phi_ref, o_ref, ob_ref = rest
        else:
            (o_ref,) = rest
        y = jnp.dot(x_ref[...], w_ref[...], preferred_element_type=jnp.float32)
        y = y + b_ref[...]
        if relu:
            y = jnp.maximum(y, 0.0)
        o_ref[...] = y
        if two:
            a = jnp.dot(y, plo_ref[...], preferred_element_type=jnp.float32)
            bb = jnp.dot(y, phi_ref[...], preferred_element_type=jnp.float32)
            ob_ref[...] = _pack_bf16_pair(a, bb)

    in_specs = [
        pl.BlockSpec((block, d), lambda i: (i, 0)),
        pl.BlockSpec((d, dout), lambda i: (0, 0)),
        pl.BlockSpec((1, dout), lambda i: (0, 0)),
    ]
    args = [x, w, b.reshape(1, dout)]
    out_specs = pl.BlockSpec((block, dout), lambda i: (i, 0))
    out_shape = jax.ShapeDtypeStruct((n, dout), jnp.float32)
    if two:
        half_spec = pl.BlockSpec((dout, dout // 2), lambda i: (0, 0))
        in_specs += [half_spec, half_spec]
        args += list(pmat)
        out_specs = [out_specs, pl.BlockSpec((block, dout // 2), lambda i: (i, 0))]
        out_shape = [out_shape, jax.ShapeDtypeStruct((n, dout // 2), jnp.int32)]

    return pl.pallas_call(
        body2,
        grid=(n // block,),
        in_specs=in_specs,
        out_specs=out_specs,
        out_shape=out_shape,
    )(*args)


def _edge_bias(bond, e_pad, wes, block):
    e, de = bond.shape
    dh = wes[0].shape[1]
    assert e_pad % block == 0
    nbi = -(-e // block)

    def body(b_ref, *rest):
        w_refs, o_refs = rest[:6], rest[6:]
        bv = b_ref[...]
        for i in range(3):
            a = jnp.dot(bv, w_refs[2 * i][...], preferred_element_type=jnp.float32)
            bb = jnp.dot(bv, w_refs[2 * i + 1][...], preferred_element_type=jnp.float32)
            o_refs[i][...] = _pack_bf16_pair(a, bb)

    w_spec = pl.BlockSpec((de, dh), lambda i: (0, 0))
    o_spec = pl.BlockSpec((block, dh), lambda i: (i, 0))
    return pl.pallas_call(
        body,
        grid=(e_pad // block,),
        in_specs=[pl.BlockSpec((block, de), lambda i: (jnp.minimum(i, nbi - 1), 0))]
        + [w_spec] * 6,
        out_specs=[o_spec, o_spec, o_spec],
        out_shape=[jax.ShapeDtypeStruct((e_pad, dh), jnp.int32)] * 3,
    )(bond, *wes)


def _dense_update(agg0, agg1, h_in, wg, bg, pmat=None, block=1000):
    n, d = h_in.shape
    assert n % block == 0
    two = pmat is not None

    def body(a0_ref, a1_ref, hin_ref, w_ref, b_ref, *rest):
        if two:
            plo_ref, phi_ref, o_ref, ob_ref = rest
        else:
            (o_ref,) = rest
        hin = hin_ref[...]
        t = a0_ref[...] + a1_ref[...] + hin
        t = jnp.dot(t, w_ref[...], preferred_element_type=jnp.float32) + b_ref[...]
        m = jnp.mean(t, axis=-1, keepdims=True)
        v = jnp.mean((t - m) * (t - m), axis=-1, keepdims=True)
        h = (t - m) * lax.rsqrt(v + 1e-5) + hin
        o_ref[...] = h
        if two:
            a = jnp.dot(h, plo_ref[...], preferred_element_type=jnp.float32)
            bb = jnp.dot(h, phi_ref[...], preferred_element_type=jnp.float32)
            ob_ref[...] = _pack_bf16_pair(a, bb)

    spec = pl.BlockSpec((block, d), lambda i: (i, 0))
    in_specs = [
        spec,
        spec,
        spec,
        pl.BlockSpec((d, d), lambda i: (0, 0)),
        pl.BlockSpec((1, d), lambda i: (0, 0)),
    ]
    args = [agg0, agg1, h_in, wg, bg.reshape(1, d)]
    out_specs = spec
    out_shape = jax.ShapeDtypeStruct((n, d), jnp.float32)
    if two:
        half_spec = pl.BlockSpec((d, d // 2), lambda i: (0, 0))
        in_specs += [half_spec, half_spec]
        args += list(pmat)
        out_specs = [spec, pl.BlockSpec((block, d // 2), lambda i: (i, 0))]
        out_shape = [out_shape, jax.ShapeDtypeStruct((n, d // 2), jnp.int32)]

    return pl.pallas_call(
        body,
        grid=(n // block,),
        in_specs=in_specs,
        out_specs=out_specs,
        out_shape=out_shape,
    )(*args)


def _dense_hin_update(agg0, agg1, h_in, wg, bg, vn, batch3, pmat, block=400):
    n, d = h_in.shape
    g = vn.shape[0]
    assert n % block == 0

    def body(b_ref, a0_ref, a1_ref, hin_ref, w_ref, bg_ref, vn_ref,
             plo_ref, phi_ref, o_ref, ob_ref):
        hin = hin_ref[...]
        t = a0_ref[...] + a1_ref[...] + hin
        t = jnp.dot(t, w_ref[...], preferred_element_type=jnp.float32) + bg_ref[...]
        m = jnp.mean(t, axis=-1, keepdims=True)
        v = jnp.mean((t - m) * (t - m), axis=-1, keepdims=True)
        h = (t - m) * lax.rsqrt(v + 1e-5) + hin
        bv = b_ref[...].reshape(block)
        onehot = (bv[:, None] == lax.broadcasted_iota(jnp.int32, (block, g), 1))
        onehot = onehot.astype(jnp.float32)
        hin2 = h + jnp.dot(onehot, vn_ref[...], preferred_element_type=jnp.float32)
        o_ref[...] = hin2
        a = jnp.dot(hin2, plo_ref[...], preferred_element_type=jnp.float32)
        bb = jnp.dot(hin2, phi_ref[...], preferred_element_type=jnp.float32)
        ob_ref[...] = _pack_bf16_pair(a, bb)

    spec = pl.BlockSpec((block, d), lambda i: (i, 0))
    half_spec = pl.BlockSpec((d, d // 2), lambda i: (0, 0))
    return pl.pallas_call(
        body,
        grid=(n // block,),
        in_specs=[
            pl.BlockSpec((1, 1, block), lambda i: (i, 0, 0)),
            spec,
            spec,
            spec,
            pl.BlockSpec((d, d), lambda i: (0, 0)),
            pl.BlockSpec((1, d), lambda i: (0, 0)),
            pl.BlockSpec((g, d), lambda i: (0, 0)),
            half_spec,
            half_spec,
        ],
        out_specs=[spec, pl.BlockSpec((block, d // 2), lambda i: (i, 0))],
        out_shape=[
            jax.ShapeDtypeStruct((n, d), jnp.float32),
            jax.ShapeDtypeStruct((n, d // 2), jnp.int32),
        ],
    )(batch3, agg0, agg1, h_in, wg, bg.reshape(1, d), vn, *pmat)


def _vn_update(h_in, vn, batch3, w1, b1, w2, b2, block=400):
    n, d = h_in.shape
    g = vn.shape[0]
    d2 = w1.shape[1]
    nb = n // block
    assert n % block == 0

    def body(b_ref, hin_ref, vn_ref, w1_ref, b1_ref, w2_ref, b2_ref, o_ref, acc):
        i = pl.program_id(0)

        @pl.when(i == 0)
        def _():
            acc[...] = jnp.zeros_like(acc)

        bv = b_ref[...].reshape(block)
        onehot = (lax.broadcasted_iota(jnp.int32, (g, block), 0) == bv[None, :])
        onehot = onehot.astype(jnp.float32)
        acc[...] += jnp.dot(onehot, hin_ref[...], preferred_element_type=jnp.float32)

        @pl.when(i == nb - 1)
        def _():
            p = acc[...] + vn_ref[...]
            t = jnp.dot(p, w1_ref[...], preferred_element_type=jnp.float32) + b1_ref[...]
            m = jnp.mean(t, axis=0, keepdims=True)
            v = jnp.mean((t - m) * (t - m), axis=0, keepdims=True)
            t = jnp.maximum((t - m) * lax.rsqrt(v + 1e-5), 0.0)
            t = jnp.dot(t, w2_ref[...], preferred_element_type=jnp.float32) + b2_ref[...]
            m = jnp.mean(t, axis=0, keepdims=True)
            v = jnp.mean((t - m) * (t - m), axis=0, keepdims=True)
            t = jnp.maximum((t - m) * lax.rsqrt(v + 1e-5), 0.0)
            o_ref[...] = vn_ref[...] + t

    return pl.pallas_call(
        body,
        grid=(nb,),
        in_specs=[
            pl.BlockSpec((1, 1, block), lambda i: (i, 0, 0)),
            pl.BlockSpec((block, d), lambda i: (i, 0)),
            pl.BlockSpec((g, d), lambda i: (0, 0)),
            pl.BlockSpec((d, d2), lambda i: (0, 0)),
            pl.BlockSpec((1, d2), lambda i: (0, 0)),
            pl.BlockSpec((d2, d), lambda i: (0, 0)),
            pl.BlockSpec((1, d), lambda i: (0, 0)),
        ],
        out_specs=pl.BlockSpec((g, d), lambda i: (0, 0)),
        out_shape=jax.ShapeDtypeStruct((g, d), jnp.float32),
        scratch_shapes=[pltpu.VMEM((g, d), jnp.float32)],
    )(batch3, h_in, vn, w1, b1.reshape(1, d2), w2, b2.reshape(1, d))


_NC = 2
_NS = 16
_CHUNK = 72


def _sc_pad_shapes(n, e):
    nw = _NC * _NS
    rows_per_tile = -(-n // _NS)
    zrows = -(-rows_per_tile // _CHUNK) * _CHUNK
    n_pad = _NS * zrows
    nchunk = -(-e // (nw * _CHUNK))
    while nchunk % 4 != 1:
        nchunk += 1
    return n_pad, nchunk


def _sc_edge_agg(hb, eb, src3, dst3):
    n = hb.shape[0]
    d = hb.shape[1] * 2
    nw, nchunk, _ = src3.shape
    ep = nchunk * _CHUNK
    assert nw == _NC * _NS
    n_pad, _ = _sc_pad_shapes(n, nw * ep)
    zrows = n_pad // _NS
    assert zrows % _CHUNK == 0 and nchunk % 4 == 1

    mesh = plsc.VectorSubcoreMesh(core_axis_name="c", subcore_axis_name="s")
    cp = pltpu.CompilerParams(
        needs_layout_passes=False, use_tc_tiling_on_sc=False
    )

    @functools.partial(
        pl.kernel,
        out_type=jax.ShapeDtypeStruct((_NC, n_pad, d), jnp.float32),
        mesh=mesh,
        compiler_params=cp,
        scratch_types=[
            [pltpu.VMEM((_CHUNK,), jnp.int32) for _ in range(4)],
            [pltpu.VMEM((_CHUNK,), jnp.int32) for _ in range(4)],
            [pltpu.VMEM((_CHUNK, d // 2), jnp.int32) for _ in range(2)],
            [pltpu.VMEM((_CHUNK, d // 2), jnp.int32) for _ in range(2)],
            [pltpu.VMEM((_CHUNK, d), jnp.float32) for _ in range(2)],
            pltpu.VMEM_SHARED((n_pad, d), jnp.float32),
            [pltpu.SemaphoreType.DMA for _ in range(4)],
            [pltpu.SemaphoreType.DMA for _ in range(2)],
            [pltpu.SemaphoreType.DMA for _ in range(2)],
            [pltpu.SemaphoreType.DMA for _ in range(2)],
        ],
    )
    def k(hb_hbm, eb_hbm, src_hbm, dst_hbm, out_hbm, sidx, didx,
          rows, ebv, obuf, acc, si, sg, se, ss):
        c = lax.axis_index("c")
        s = lax.axis_index("s")
        wid = s * _NC + c
        ebase = wid * ep

        def issue_idx(ci, q):
            pltpu.async_copy(src_hbm.at[wid, ci], sidx[q], si[q])
            pltpu.async_copy(dst_hbm.at[wid, ci], didx[q], si[q])

        def wait_idx(q):
            pltpu.make_async_copy(src_hbm.at[wid, 0], sidx[q], si[q]).wait()
            pltpu.make_async_copy(dst_hbm.at[wid, 0], didx[q], si[q]).wait()

        def issue_gather(ci, p, q):
            pltpu.async_copy(hb_hbm.at[sidx[q]], rows[p], sg[p])
            pltpu.async_copy(eb_hbm.at[pl.ds(ebase + ci * _CHUNK, _CHUNK)],
                             ebv[p], se[p])

        def wait_gather(p):
            pltpu.make_async_copy(hb_hbm.at[sidx[0]], rows[p], sg[p]).wait()
            pltpu.make_async_copy(eb_hbm.at[pl.ds(ebase, _CHUNK)],
                                  ebv[p], se[p]).wait()

        def wait_scat(p):
            pltpu.make_async_copy(obuf[p], acc.at[didx[0]], ss[p]).wait()

        def compute(p):
            rp = rows[p]
            ep_ = ebv[p]
            op = obuf[p]

            @pl.loop(0, _CHUNK, unroll=2)
            def _(r):
                for j in range(d // 32):
                    slw = pl.ds(j * 16, 16)
                    a = plsc.bitcast(rp[r, slw], jnp.bfloat16)
                    b = plsc.bitcast(ep_[r, slw], jnp.bfloat16)
                    m = jnp.maximum(a + b, jnp.bfloat16(0))
                    w = plsc.bitcast(m, jnp.int32)
                    op[r, pl.ds(j * 32, 16)] = plsc.bitcast(w << 16, jnp.float32)
                    op[r, pl.ds(j * 32 + 16, 16)] = plsc.bitcast(w, jnp.float32)

        issue_idx(0, 0)
        issue_idx(1, 1)

        @pl.loop(0, _CHUNK)
        def _(r):
            for j in range(d // 16):
                obuf[0][r, pl.ds(j * 16, 16)] = jnp.zeros((16,), jnp.float32)

        @pl.loop(0, zrows, step=_CHUNK)
        def _(r0):
            pltpu.sync_copy(obuf[0], acc.at[pl.ds(s * zrows + r0, _CHUNK)])

        plsc.subcore_barrier()

        wait_idx(0)
        issue_gather(0, 0, 0)
        wait_idx(1)
        issue_gather(1, 1, 1)

        def step(ci, u):
            p, q, q2 = u % 2, u % 4, (u + 2) % 4
            cc = ci + u
            wait_gather(p)

            @pl.when(cc >= 2)
            def _():
                wait_scat(p)

            @pl.when(cc + 2 < nchunk)
            def _():
                issue_idx(cc + 2, q2)

            compute(p)
            pltpu.async_copy(obuf[p], acc.at[didx[q]], ss[p], add=True)

            @pl.when(cc + 2 < nchunk)
            def _():
                wait_idx(q2)
                issue_gather(cc + 2, p, q2)

        @pl.loop(0, nchunk - 1, step=4)
        def _(ci):
            for u in range(4):
                step(ci, u)

        wait_gather(0)
        wait_scat(0)
        compute(0)
        pltpu.async_copy(obuf[0], acc.at[didx[0]], ss[0], add=True)
        wait_scat(0)
        wait_scat(1)

        plsc.subcore_barrier()
        pltpu.sync_copy(
            acc.at[pl.ds(s * zrows, zrows)], out_hbm.at[c, pl.ds(s * zrows, zrows)]
        )

    return k(hb, eb, src3, dst3)


def kernel(x, edge_index, bond_feature, edge_attr, peripheral_attr, rd, batch,
           W_init, b_init, We0, Wg0, bg0, We1, Wg1, bg1, We2, Wg2, bg2,
           Wv1_0, bv1_0, Wv2_0, bv2_0, Wv1_1, bv1_1, Wv2_1, bv2_1,
           W_out, b_out):
    n, d = x.shape
    g = 512
    nw = _NC * _NS
    e = edge_index.shape[1]
    n_pad, nchunk = _sc_pad_shapes(n, e)
    e_pad = nw * nchunk * _CHUNK
    npad_e = e_pad - e
    ar = jnp.arange(npad_e, dtype=jnp.int32)
    src_p = jnp.concatenate([edge_index[0], ar % n])
    dst_p = jnp.concatenate([edge_index[1], n + ar % (n_pad - n)])
    src3 = src_p.reshape(nw, nchunk, _CHUNK)
    dst3 = dst_p.reshape(nw, nchunk, _CHUNK)
    batch3 = batch.reshape(n // 400, 1, 400)

    lo_idx, hi_idx = _lohi_idx(d)
    eye = jnp.eye(d, dtype=jnp.float32)
    pmat = (eye[:, lo_idx], eye[:, hi_idx])

    h0, hb = _mm_bias(x, W_init, b_init, pmat=pmat)
    wes = (We0[:, lo_idx], We0[:, hi_idx], We1[:, lo_idx], We1[:, hi_idx],
           We2[:, lo_idx], We2[:, hi_idx])
    eb0, eb1, eb2 = _edge_bias(bond_feature, e_pad, wes, block=e_pad // nchunk)

    wgs = (Wg0, Wg1, Wg2)
    bgs = (bg0, bg1, bg2)
    ebs = (eb0, eb1, eb2)
    wv1 = (Wv1_0, Wv1_1)
    bv1 = (bv1_0, bv1_1)
    wv2 = (Wv2_0, Wv2_1)
    bv2 = (bv2_0, bv2_1)

    vn = jnp.zeros((g, d), dtype=jnp.float32)
    h_in = h0
    for l in range(3):
        agg = _sc_edge_agg(hb, ebs[l], src3, dst3)
        if l < 2:
            vn = _vn_update(h_in, vn, batch3, wv1[l], bv1[l], wv2[l], bv2[l])
            h_in, hb = _dense_hin_update(
                agg[0], agg[1], h_in, wgs[l], bgs[l], vn, batch3, pmat
            )
        else:
            h_in = _dense_update(agg[0], agg[1], h_in, wgs[l], bgs[l])

    return _mm_bias(h_in, W_out, b_out, relu=True)


# speedup vs baseline: 2.7419x; 1.0256x over previous
import dataclasses
import functools

import jax
import jax.numpy as jnp
from jax import lax
from jax.experimental import pallas as pl
from jax.experimental.pallas import tpu as pltpu
from jax.experimental.pallas import tpu_sc as plsc


def _lohi_idx(d):
    import numpy as np

    w = np.arange(d // 2)
    lo = 32 * (w // 16) + w % 16
    return lo, lo + 16


def _pack_bf16_pair(a, b):
    ua = lax.bitcast_convert_type(a, jnp.uint32)
    ub = lax.bitcast_convert_type(b, jnp.uint32)
    one = jnp.uint32(1)
    half = jnp.uint32(0x7FFF)
    bfa = (ua + half + ((ua >> 16) & one)) >> 16
    bfb = (ub + half + ((ub >> 16) & one)) >> 16
    return lax.bitcast_convert_type(bfa | (bfb << 16), jnp.int32)


def _mm_bias(x, w, b, pmat=None, relu=False, block=1000):
    n, d = x.shape
    dout = w.shape[1]
    assert n % block == 0

    two = pmat is not None

    def body2(x_ref, w_ref, b_ref, *rest):
        if two:
            plo_ref, phi_ref, o_ref, ob_ref = rest
        else:
            (o_ref,) = rest
        y = jnp.dot(x_ref[...], w_ref[...], preferred_element_type=jnp.float32)
        y = y + b_ref[...]
        if relu:
            y = jnp.maximum(y, 0.0)
        o_ref[...] = y
        if two:
            yb = y.astype(jnp.bfloat16)
            a = jnp.dot(yb, plo_ref[...], preferred_element_type=jnp.float32)
            bb = jnp.dot(yb, phi_ref[...], preferred_element_type=jnp.float32)
            ob_ref[...] = _pack_bf16_pair(a, bb)

    in_specs = [
        pl.BlockSpec((block, d), lambda i: (i, 0)),
        pl.BlockSpec((d, dout), lambda i: (0, 0)),
        pl.BlockSpec((1, dout), lambda i: (0, 0)),
    ]
    args = [x, w, b.reshape(1, dout)]
    out_specs = pl.BlockSpec((block, dout), lambda i: (i, 0))
    out_shape = jax.ShapeDtypeStruct((n, dout), jnp.float32)
    if two:
        half_spec = pl.BlockSpec((dout, dout // 2), lambda i: (0, 0))
        in_specs += [half_spec, half_spec]
        args += list(pmat)
        out_specs = [out_specs, pl.BlockSpec((block, dout // 2), lambda i: (i, 0))]
        out_shape = [out_shape, jax.ShapeDtypeStruct((n, dout // 2), jnp.int32)]

    return pl.pallas_call(
        body2,
        grid=(n // block,),
        in_specs=in_specs,
        out_specs=out_specs,
        out_shape=out_shape,
    )(*args)


def _edge_bias(bond, e_pad, wcat, block):
    e, de = bond.shape
    dh = wcat.shape[1] // 6
    assert e_pad % block == 0
    nbi = -(-e // block)

    def body(b_ref, w_ref, o0_ref, o1_ref, o2_ref):
        y = jnp.dot(b_ref[...].astype(jnp.bfloat16), w_ref[...],
                    preferred_element_type=jnp.float32)
        for i, o_ref in enumerate((o0_ref, o1_ref, o2_ref)):
            a = y[:, 2 * i * dh:(2 * i + 1) * dh]
            bb = y[:, (2 * i + 1) * dh:(2 * i + 2) * dh]
            o_ref[...] = _pack_bf16_pair(a, bb)

    o_spec = pl.BlockSpec((block, dh), lambda i: (i, 0))
    return pl.pallas_call(
        body,
        grid=(e_pad // block,),
        in_specs=[
            pl.BlockSpec((block, de), lambda i: (jnp.minimum(i, nbi - 1), 0)),
            pl.BlockSpec((de, 6 * dh), lambda i: (0, 0)),
        ],
        out_specs=[o_spec, o_spec, o_spec],
        out_shape=[jax.ShapeDtypeStruct((e_pad, dh), jnp.int32)] * 3,
    )(bond, wcat)


def _dense_update(agg0, agg1, h_in, wg, bg, pmat=None, block=1000):
    n, d = h_in.shape
    assert n % block == 0
    two = pmat is not None

    def body(a0_ref, a1_ref, hin_ref, w_ref, b_ref, *rest):
        if two:
            plo_ref, phi_ref, o_ref, ob_ref = rest
        else:
            (o_ref,) = rest
        hin = hin_ref[...]
        t = a0_ref[...] + a1_ref[...] + hin
        t = jnp.dot(t, w_ref[...], preferred_element_type=jnp.float32) + b_ref[...]
        m = jnp.mean(t, axis=-1, keepdims=True)
        v = jnp.mean((t - m) * (t - m), axis=-1, keepdims=True)
        h = (t - m) * lax.rsqrt(v + 1e-5) + hin
        o_ref[...] = h
        if two:
            a = jnp.dot(h, plo_ref[...], preferred_element_type=jnp.float32)
            bb = jnp.dot(h, phi_ref[...], preferred_element_type=jnp.float32)
            ob_ref[...] = _pack_bf16_pair(a, bb)

    spec = pl.BlockSpec((block, d), lambda i: (i, 0))
    in_specs = [
        spec,
        spec,
        spec,
        pl.BlockSpec((d, d), lambda i: (0, 0)),
        pl.BlockSpec((1, d), lambda i: (0, 0)),
    ]
    args = [agg0, agg1, h_in, wg, bg.reshape(1, d)]
    out_specs = spec
    out_shape = jax.ShapeDtypeStruct((n, d), jnp.float32)
    if two:
        half_spec = pl.BlockSpec((d, d // 2), lambda i: (0, 0))
        in_specs += [half_spec, half_spec]
        args += list(pmat)
        out_specs = [spec, pl.BlockSpec((block, d // 2), lambda i: (i, 0))]
        out_shape = [out_shape, jax.ShapeDtypeStruct((n, d // 2), jnp.int32)]

    return pl.pallas_call(
        body,
        grid=(n // block,),
        in_specs=in_specs,
        out_specs=out_specs,
        out_shape=out_shape,
    )(*args)


def _dense_hin_update(agg0, agg1, h_in, wg, bg, vn, batch3, pmat, block=400):
    n, d = h_in.shape
    g = vn.shape[0]
    assert n % block == 0

    def body(b_ref, a0_ref, a1_ref, hin_ref, w_ref, bg_ref, vn_ref,
             plo_ref, phi_ref, o_ref, ob_ref):
        hin = hin_ref[...]
        t = a0_ref[...] + a1_ref[...] + hin
        t = jnp.dot(t, w_ref[...], preferred_element_type=jnp.float32) + bg_ref[...]
        m = jnp.mean(t, axis=-1, keepdims=True)
        v = jnp.mean((t - m) * (t - m), axis=-1, keepdims=True)
        h = (t - m) * lax.rsqrt(v + 1e-5) + hin
        bv = b_ref[...].reshape(block)
        onehot = (bv[:, None] == lax.broadcasted_iota(jnp.int32, (block, g), 1))
        onehot = onehot.astype(jnp.float32)
        hin2 = h + jnp.dot(onehot, vn_ref[...], preferred_element_type=jnp.float32)
        o_ref[...] = hin2
        hb2 = hin2.astype(jnp.bfloat16)
        a = jnp.dot(hb2, plo_ref[...], preferred_element_type=jnp.float32)
        bb = jnp.dot(hb2, phi_ref[...], preferred_element_type=jnp.float32)
        ob_ref[...] = _pack_bf16_pair(a, bb)

    spec = pl.BlockSpec((block, d), lambda i: (i, 0))
    half_spec = pl.BlockSpec((d, d // 2), lambda i: (0, 0))
    return pl.pallas_call(
        body,
        grid=(n // block,),
        in_specs=[
            pl.BlockSpec((1, 1, block), lambda i: (i, 0, 0)),
            spec,
            spec,
            spec,
            pl.BlockSpec((d, d), lambda i: (0, 0)),
            pl.BlockSpec((1, d), lambda i: (0, 0)),
            pl.BlockSpec((g, d), lambda i: (0, 0)),
            half_spec,
            half_spec,
        ],
        out_specs=[spec, pl.BlockSpec((block, d // 2), lambda i: (i, 0))],
        out_shape=[
            jax.ShapeDtypeStruct((n, d), jnp.float32),
            jax.ShapeDtypeStruct((n, d // 2), jnp.int32),
        ],
    )(batch3, agg0, agg1, h_in, wg, bg.reshape(1, d), vn, *pmat)


def _vn_update(h_in, vn, batch3, w1, b1, w2, b2, block=400):
    n, d = h_in.shape
    g = vn.shape[0]
    d2 = w1.shape[1]
    nb = n // block
    assert n % block == 0

    def body(b_ref, hin_ref, vn_ref, w1_ref, b1_ref, w2_ref, b2_ref, o_ref, acc):
        i = pl.program_id(0)

        @pl.when(i == 0)
        def _():
            acc[...] = jnp.zeros_like(acc)

        bv = b_ref[...].reshape(block)
        onehot = (lax.broadcasted_iota(jnp.int32, (g, block), 0) == bv[None, :])
        onehot = onehot.astype(jnp.float32)
        acc[...] += jnp.dot(onehot, hin_ref[...], preferred_element_type=jnp.float32)

        @pl.when(i == nb - 1)
        def _():
            p = acc[...] + vn_ref[...]
            t = jnp.dot(p, w1_ref[...], preferred_element_type=jnp.float32) + b1_ref[...]
            m = jnp.mean(t, axis=0, keepdims=True)
            v = jnp.mean((t - m) * (t - m), axis=0, keepdims=True)
            t = jnp.maximum((t - m) * lax.rsqrt(v + 1e-5), 0.0)
            t = jnp.dot(t, w2_ref[...], preferred_element_type=jnp.float32) + b2_ref[...]
            m = jnp.mean(t, axis=0, keepdims=True)
            v = jnp.mean((t - m) * (t - m), axis=0, keepdims=True)
            t = jnp.maximum((t - m) * lax.rsqrt(v + 1e-5), 0.0)
            o_ref[...] = vn_ref[...] + t

    return pl.pallas_call(
        body,
        grid=(nb,),
        in_specs=[
            pl.BlockSpec((1, 1, block), lambda i: (i, 0, 0)),
            pl.BlockSpec((block, d), lambda i: (i, 0)),
            pl.BlockSpec((g, d), lambda i: (0, 0)),
            pl.BlockSpec((d, d2), lambda i: (0, 0)),
            pl.BlockSpec((1, d2), lambda i: (0, 0)),
            pl.BlockSpec((d2, d), lambda i: (0, 0)),
            pl.BlockSpec((1, d), lambda i: (0, 0)),
        ],
        out_specs=pl.BlockSpec((g, d), lambda i: (0, 0)),
        out_shape=jax.ShapeDtypeStruct((g, d), jnp.float32),
        scratch_shapes=[pltpu.VMEM((g, d), jnp.float32)],
    )(batch3, h_in, vn, w1, b1.reshape(1, d2), w2, b2.reshape(1, d))


_NC = 2
_NS = 16
_CHUNK = 80


def _sc_pad_shapes(n, e):
    nw = _NC * _NS
    rows_per_tile = -(-n // _NS)
    zrows = -(-rows_per_tile // _CHUNK) * _CHUNK
    n_pad = _NS * zrows
    nchunk = -(-e // (nw * _CHUNK))
    while nchunk % 4 != 1:
        nchunk += 1
    return n_pad, nchunk


def _sc_edge_agg(hb, eb, src3, dst3):
    n = hb.shape[0]
    d = hb.shape[1] * 2
    nw, nchunk, _ = src3.shape
    ep = nchunk * _CHUNK
    assert nw == _NC * _NS
    n_pad, _ = _sc_pad_shapes(n, nw * ep)
    zrows = n_pad // _NS
    assert zrows % _CHUNK == 0 and nchunk % 4 == 1

    mesh = plsc.VectorSubcoreMesh(core_axis_name="c", subcore_axis_name="s")
    cp = pltpu.CompilerParams(
        needs_layout_passes=False, use_tc_tiling_on_sc=False
    )

    @functools.partial(
        pl.kernel,
        out_type=jax.ShapeDtypeStruct((_NC, n_pad, d), jnp.float32),
        mesh=mesh,
        compiler_params=cp,
        scratch_types=[
            [pltpu.VMEM((_CHUNK,), jnp.int32) for _ in range(4)],
            [pltpu.VMEM((_CHUNK,), jnp.int32) for _ in range(4)],
            [pltpu.VMEM((_CHUNK, d // 2), jnp.int32) for _ in range(2)],
            [pltpu.VMEM((_CHUNK, d // 2), jnp.int32) for _ in range(2)],
            [pltpu.VMEM((_CHUNK, d), jnp.float32) for _ in range(2)],
            pltpu.VMEM_SHARED((n_pad, d), jnp.float32),
            [pltpu.SemaphoreType.DMA for _ in range(4)],
            [pltpu.SemaphoreType.DMA for _ in range(2)],
            [pltpu.SemaphoreType.DMA for _ in range(2)],
            [pltpu.SemaphoreType.DMA for _ in range(2)],
        ],
    )
    def k(hb_hbm, eb_hbm, src_hbm, dst_hbm, out_hbm, sidx, didx,
          rows, ebv, obuf, acc, si, sg, se, ss):
        c = lax.axis_index("c")
        s = lax.axis_index("s")
        wid = s * _NC + c
        ebase = wid * ep

        def issue_idx(ci, q):
            pltpu.async_copy(src_hbm.at[wid, ci], sidx[q], si[q])
            pltpu.async_copy(dst_hbm.at[wid, ci], didx[q], si[q])

        def wait_idx(q):
            pltpu.make_async_copy(src_hbm.at[wid, 0], sidx[q], si[q]).wait()
            pltpu.make_async_copy(dst_hbm.at[wid, 0], didx[q], si[q]).wait()

        def issue_gather(ci, p, q):
            pltpu.async_copy(hb_hbm.at[sidx[q]], rows[p], sg[p])
            pltpu.async_copy(eb_hbm.at[pl.ds(ebase + ci * _CHUNK, _CHUNK)],
                             ebv[p], se[p])

        def wait_gather(p):
            pltpu.make_async_copy(hb_hbm.at[sidx[0]], rows[p], sg[p]).wait()
            pltpu.make_async_copy(eb_hbm.at[pl.ds(ebase, _CHUNK)],
                                  ebv[p], se[p]).wait()

        def wait_scat(p):
            pltpu.make_async_copy(obuf[p], acc.at[didx[0]], ss[p]).wait()

        def compute(p):
            rp = rows[p]
            ep_ = ebv[p]
            op = obuf[p]

            @pl.loop(0, _CHUNK, unroll=2)
            def _(r):
                for j in range(d // 32):
                    slw = pl.ds(j * 16, 16)
                    a = plsc.bitcast(rp[r, slw], jnp.bfloat16)
                    b = plsc.bitcast(ep_[r, slw], jnp.bfloat16)
                    m = jnp.maximum(a + b, jnp.bfloat16(0))
                    w = plsc.bitcast(m, jnp.int32)
                    op[r, pl.ds(j * 32, 16)] = plsc.bitcast(w << 16, jnp.float32)
                    op[r, pl.ds(j * 32 + 16, 16)] = plsc.bitcast(w, jnp.float32)

        issue_idx(0, 0)
        issue_idx(1, 1)

        @pl.loop(0, _CHUNK)
        def _(r):
            for j in range(d // 16):
                obuf[0][r, pl.ds(j * 16, 16)] = jnp.zeros((16,), jnp.float32)

        @pl.loop(0, zrows, step=_CHUNK)
        def _(r0):
            pltpu.sync_copy(obuf[0], acc.at[pl.ds(s * zrows + r0, _CHUNK)])

        plsc.subcore_barrier()

        wait_idx(0)
        issue_gather(0, 0, 0)
        wait_idx(1)
        issue_gather(1, 1, 1)

        def step(ci, u):
            p, q, q2 = u % 2, u % 4, (u + 2) % 4
            cc = ci + u
            wait_gather(p)

            @pl.when(cc >= 2)
            def _():
                wait_scat(p)

            @pl.when(cc + 2 < nchunk)
            def _():
                issue_idx(cc + 2, q2)

            compute(p)
            pltpu.async_copy(obuf[p], acc.at[didx[q]], ss[p], add=True)

            @pl.when(cc + 2 < nchunk)
            def _():
                wait_idx(q2)
                issue_gather(cc + 2, p, q2)

        @pl.loop(0, nchunk - 1, step=4)
        def _(ci):
            for u in range(4):
                step(ci, u)

        wait_gather(0)
        wait_scat(0)
        compute(0)
        pltpu.async_copy(obuf[0], acc.at[didx[0]], ss[0], add=True)
        wait_scat(0)
        wait_scat(1)

        plsc.subcore_barrier()
        pltpu.sync_copy(
            acc.at[pl.ds(s * zrows, zrows)], out_hbm.at[c, pl.ds(s * zrows, zrows)]
        )

    return k(hb, eb, src3, dst3)


def kernel(x, edge_index, bond_feature, edge_attr, peripheral_attr, rd, batch,
           W_init, b_init, We0, Wg0, bg0, We1, Wg1, bg1, We2, Wg2, bg2,
           Wv1_0, bv1_0, Wv2_0, bv2_0, Wv1_1, bv1_1, Wv2_1, bv2_1,
           W_out, b_out):
    n, d = x.shape
    g = 512
    nw = _NC * _NS
    e = edge_index.shape[1]
    n_pad, nchunk = _sc_pad_shapes(n, e)
    e_pad = nw * nchunk * _CHUNK
    npad_e = e_pad - e
    ar = jnp.arange(npad_e, dtype=jnp.int32)
    src_p = jnp.concatenate([edge_index[0], ar % n])
    dst_p = jnp.concatenate([edge_index[1], n + ar % (n_pad - n)])
    src3 = src_p.reshape(nw, nchunk, _CHUNK)
    dst3 = dst_p.reshape(nw, nchunk, _CHUNK)
    batch3 = batch.reshape(n // 400, 1, 400)

    lo_idx, hi_idx = _lohi_idx(d)
    eye = jnp.eye(d, dtype=jnp.bfloat16)
    pmat = (eye[:, lo_idx], eye[:, hi_idx])

    h0, hb = _mm_bias(x, W_init, b_init, pmat=pmat)
    wcat = jnp.concatenate(
        [We0[:, lo_idx], We0[:, hi_idx], We1[:, lo_idx], We1[:, hi_idx],
         We2[:, lo_idx], We2[:, hi_idx]], axis=1).astype(jnp.bfloat16)
    eb0, eb1, eb2 = _edge_bias(bond_feature, e_pad, wcat, block=e_pad // nchunk)

    wgs = (Wg0, Wg1, Wg2)
    bgs = (bg0, bg1, bg2)
    ebs = (eb0, eb1, eb2)
    wv1 = (Wv1_0, Wv1_1)
    bv1 = (bv1_0, bv1_1)
    wv2 = (Wv2_0, Wv2_1)
    bv2 = (bv2_0, bv2_1)

    vn = jnp.zeros((g, d), dtype=jnp.float32)
    h_in = h0
    for l in range(3):
        agg = _sc_edge_agg(hb, ebs[l], src3, dst3)
        if l < 2:
            vn = _vn_update(h_in, vn, batch3, wv1[l], bv1[l], wv2[l], bv2[l])
            h_in, hb = _dense_hin_update(
                agg[0], agg[1], h_in, wgs[l], bgs[l], vn, batch3, pmat
            )
        else:
            h_in = _dense_update(agg[0], agg[1], h_in, wgs[l], bgs[l])

    return _mm_bias(h_in, W_out, b_out, relu=True)
